# baseline (speedup 1.0000x reference)
import functools

import jax
import jax.numpy as jnp
from jax import lax
from jax.experimental import pallas as pl
from jax.experimental.pallas import tpu as pltpu
from jax.experimental.pallas import tpu_sc as plsc

N = 10000
E = 320000
D = 128
H = 128
C = 64

N_PAD = 10112
ROW_BLK = 1264

NC = 2
NS = 16
BLK = 64
PH = 40
P0_L1, P1_L1 = 8, 0
P0_L2, P1_L2 = 6, 2
EPP = NS * PH * BLK
E_PAD = 8 * EPP
STRIPE = N_PAD // NS


def _stage_a_body(x_ref, w1lt_ref, w1rt_ref, xt1_ref, xr1_ref):
    x = x_ref[...]
    xt1_ref[...] = jnp.dot(x, w1lt_ref[...], preferred_element_type=jnp.float32)
    xr1_ref[...] = jnp.dot(x, w1rt_ref[...], preferred_element_type=jnp.float32)


def _stage_b_body(s1a_ref, s1b_ref, cnta_ref, cntb_ref, xr1_ref, b1l_ref,
                  w2lt_ref, w2rt_ref, ht2_ref, hr2_ref):
    cnt = cnta_ref[:, 0:1] + cntb_ref[:, 0:1]
    den = jnp.maximum(cnt, 1.0)
    h = (s1a_ref[...] + s1b_ref[...]) / den + b1l_ref[...] + xr1_ref[...]
    h = jnp.maximum(h, 0.0)
    ht2_ref[...] = jnp.dot(h, w2lt_ref[...], preferred_element_type=jnp.float32)
    hr2_ref[...] = jnp.dot(h, w2rt_ref[...], preferred_element_type=jnp.float32)


def _stage_c_body(s2a_ref, s2b_ref, cnta_ref, cntb_ref, hr2_ref, b2l_ref, out_ref):
    cnt = cnta_ref[:, 0:1] + cntb_ref[:, 0:1]
    den = jnp.maximum(cnt, 1.0)
    o = (s2a_ref[...] + s2b_ref[...]) / den + b2l_ref[...] + hr2_ref[...]
    m = jnp.max(o, axis=1, keepdims=True)
    ex = jnp.exp(o - m)
    lse = m + jnp.log(jnp.sum(ex, axis=1, keepdims=True))
    out_ref[...] = o - lse


def _row_spec(w):
    return pl.BlockSpec((ROW_BLK, w), lambda i: (i, 0))


def _full_spec(h, w):
    return pl.BlockSpec((h, w), lambda i: (0, 0))


def _stage_a(x_pad, w1lt, w1rt):
    return pl.pallas_call(
        _stage_a_body,
        grid=(N_PAD // ROW_BLK,),
        in_specs=[_row_spec(D), _full_spec(D, H), _full_spec(D, H)],
        out_specs=[_row_spec(H), _row_spec(H)],
        out_shape=[jax.ShapeDtypeStruct((N_PAD, H), jnp.float32)] * 2,
    )(x_pad, w1lt, w1rt)


def _stage_b(s1a, s1b, cnta, cntb, xr1, b1l, w2lt, w2rt):
    return pl.pallas_call(
        _stage_b_body,
        grid=(N_PAD // ROW_BLK,),
        in_specs=[_row_spec(H), _row_spec(H), _row_spec(16), _row_spec(16),
                  _row_spec(H), _full_spec(1, H), _full_spec(H, C), _full_spec(H, C)],
        out_specs=[_row_spec(C), _row_spec(C)],
        out_shape=[jax.ShapeDtypeStruct((N_PAD, C), jnp.float32)] * 2,
    )(s1a, s1b, cnta, cntb, xr1, b1l, w2lt, w2rt)


def _stage_c(s2a, s2b, cnta, cntb, hr2, b2l):
    return pl.pallas_call(
        _stage_c_body,
        grid=(N_PAD // ROW_BLK,),
        in_specs=[_row_spec(C), _row_spec(C), _row_spec(16), _row_spec(16),
                  _row_spec(C), _full_spec(1, C)],
        out_specs=_row_spec(C),
        out_shape=jax.ShapeDtypeStruct((N_PAD, C), jnp.float32),
    )(s2a, s2b, cnta, cntb, hr2, b2l)


def _make_sc_seg_sum(width, with_cnt, p0, p1):
    mesh = plsc.VectorSubcoreMesh(core_axis_name="c", subcore_axis_name="s",
                                  num_cores=NC, num_subcores=NS)
    out_type = [jax.ShapeDtypeStruct((N_PAD, width), jnp.float32)] * NC
    if with_cnt:
        out_type += [jax.ShapeDtypeStruct((N_PAD, 16), jnp.float32)] * NC
    scratch = [
        pltpu.VMEM((PH, BLK), jnp.int32),
        pltpu.VMEM((PH, BLK), jnp.int32),
        pltpu.VMEM((BLK, width), jnp.float32),
        pltpu.VMEM((BLK, width), jnp.float32),
        pltpu.SemaphoreType.DMA,
        pltpu.SemaphoreType.DMA,
        pltpu.SemaphoreType.DMA,
        pltpu.SemaphoreType.DMA,
        pltpu.SemaphoreType.DMA,
        pltpu.VMEM_SHARED((N_PAD, width), jnp.float32),
    ]
    if with_cnt:
        scratch += [
            pltpu.VMEM((BLK, 16), jnp.float32),
            pltpu.VMEM_SHARED((N_PAD, 16), jnp.float32),
        ]

    def body(*refs):
        ni = 4 if p1 else 2
        if with_cnt:
            table = refs[0]
            idxs = refs[1:1 + ni]
            (zerosw, zeros16, ones16) = refs[1 + ni:4 + ni]
            outs = refs[4 + ni:4 + ni + NC]
            cnts = refs[4 + ni + NC:4 + ni + 2 * NC]
            (src_v, dst_v, rows0, rows1, g0, g1, s0, s1, sc_sem,
             accum_sh, ones_v, cnt_sh) = refs[4 + ni + 2 * NC:]
        else:
            table = refs[0]
            idxs = refs[1:1 + ni]
            zerosw = refs[1 + ni]
            outs = refs[2 + ni:2 + ni + NC]
            (src_v, dst_v, rows0, rows1, g0, g1, s0, s1, sc_sem,
             accum_sh) = refs[2 + ni + NC:]
        if p1:
            srcA, srcB, dstA, dstB = idxs
        else:
            (srcA, dstA), srcB, dstB = idxs, None, None
        cid = lax.axis_index("c")
        sid = lax.axis_index("s")
        base = sid * STRIPE
        stripe = pl.ds(base, STRIPE)

        pltpu.sync_copy(zerosw.at[stripe], accum_sh.at[stripe])
        if with_cnt:
            pltpu.sync_copy(ones16, ones_v)
            pltpu.sync_copy(zeros16.at[stripe], cnt_sh.at[stripe])
        plsc.subcore_barrier()

        rows = (rows0, rows1)
        gsem = (g0, g1)
        ssem = (s0, s1)

        def gather(j, b):
            pltpu.async_copy(table.at[src_v.at[j]], rows[b], gsem[b])

        def wait_gather(j, b):
            pltpu.make_async_copy(table.at[src_v.at[j]], rows[b], gsem[b]).wait()

        def scatter(j, b):
            pltpu.async_copy(rows[b], accum_sh.at[dst_v.at[j]], ssem[b], add=True)
            if with_cnt:
                pltpu.async_copy(ones_v, cnt_sh.at[dst_v.at[j]], sc_sem, add=True)

        def wait_scatter(j, b):
            pltpu.make_async_copy(rows[b], accum_sh.at[dst_v.at[j]], ssem[b]).wait()

        def wait_cnt(j):
            if with_cnt:
                pltpu.make_async_copy(ones_v, cnt_sh.at[dst_v.at[j]], sc_sem).wait()

        def phase(srcr, dstr, p):
            pltpu.sync_copy(srcr.at[sid, pl.ds(p * PH, PH)], src_v)
            pltpu.sync_copy(dstr.at[sid, pl.ds(p * PH, PH)], dst_v)
            gather(0, 0)

            def pair(q, carry):
                j = 2 * q

                @pl.when(q > 0)
                def _():
                    wait_scatter(j - 1, 1)
                    wait_cnt(j - 2)
                    wait_cnt(j - 1)
                gather(j + 1, 1)
                wait_gather(j, 0)
                scatter(j, 0)

                @pl.when(q < PH // 2 - 1)
                def _():
                    wait_scatter(j, 0)
                    gather(j + 2, 0)
                wait_gather(j + 1, 1)
                scatter(j + 1, 1)
                return carry

            lax.fori_loop(0, PH // 2, pair, 0)
            wait_scatter(PH - 2, 0)
            wait_cnt(PH - 2)
            wait_scatter(PH - 1, 1)
            wait_cnt(PH - 1)

        @pl.when(cid == 0)
        def _():
            for p in range(p0):
                phase(srcA, dstA, p)

        if p1:
            @pl.when(cid == 1)
            def _():
                for p in range(p1):
                    phase(srcB, dstB, p)
        plsc.subcore_barrier()

        for c in range(NC):
            @pl.when(cid == c)
            def _(c=c):
                pltpu.sync_copy(accum_sh.at[stripe], outs[c].at[stripe])
                if with_cnt:
                    pltpu.sync_copy(cnt_sh.at[stripe], cnts[c].at[stripe])

    return pl.kernel(
        body, out_type=out_type, mesh=mesh, scratch_types=scratch,
        compiler_params=pltpu.CompilerParams(use_tc_tiling_on_sc=False))


_seg_sum_l1 = _make_sc_seg_sum(H, True, P0_L1, P1_L1)
_seg_sum_l2 = _make_sc_seg_sum(C, False, P0_L2, P1_L2)


def _split_edges(arr, fill, p0, p1):
    e0 = min(NS * p0 * PH * BLK, E)
    pad0 = NS * p0 * PH * BLK - e0
    a = jnp.concatenate([arr[:e0], jnp.full((pad0,), fill, jnp.int32)])
    a = a.reshape(NS, p0 * PH, BLK)
    if p1 == 0:
        return (a,)
    pad1 = NS * p1 * PH * BLK - (E - e0)
    b = jnp.concatenate([arr[e0:], jnp.full((pad1,), fill, jnp.int32)])
    return a, b.reshape(NS, p1 * PH, BLK)


def kernel(x, edge_index, W1l, b1l, W1r, W2l, b2l, W2r):
    src = edge_index[0]
    dst = edge_index[1]

    s1_idx = _split_edges(src, 0, P0_L1, P1_L1) + _split_edges(dst, N, P0_L1, P1_L1)
    s2_idx = _split_edges(src, 0, P0_L2, P1_L2) + _split_edges(dst, N, P0_L2, P1_L2)

    x_pad = jnp.zeros((N_PAD, D), jnp.float32).at[:N].set(x)
    w1lt = W1l.T
    w1rt = W1r.T
    w2lt = W2l.T
    w2rt = W2r.T

    zeros_h = jnp.zeros((N_PAD, H), jnp.float32)
    zeros_c = jnp.zeros((N_PAD, C), jnp.float32)
    zeros16 = jnp.zeros((N_PAD, 16), jnp.float32)
    ones16 = jnp.ones((BLK, 16), jnp.float32)

    xt1, xr1 = _stage_a(x_pad, w1lt, w1rt)

    r1 = _seg_sum_l1(xt1, *s1_idx, zeros_h, zeros16, ones16)
    if NC == 2:
        s1a, s1b, cnta, cntb = r1
    else:
        (s1a, cnta), s1b, cntb = r1, zeros_h, zeros16

    ht2, hr2 = _stage_b(s1a, s1b, cnta, cntb, xr1, b1l.reshape(1, H), w2lt, w2rt)

    r2 = _seg_sum_l2(ht2, *s2_idx, zeros_c)
    if NC == 2:
        s2a, s2b = r2
    else:
        (s2a,), s2b = r2, zeros_c

    out = _stage_c(s2a, s2b, cnta, cntb, hr2, b2l.reshape(1, C))
    return out[:N]


# speedup vs baseline: 1.1532x; 1.1532x over previous
import functools

import jax
import jax.numpy as jnp
from jax import lax
from jax.experimental import pallas as pl
from jax.experimental.pallas import tpu as pltpu
from jax.experimental.pallas import tpu_sc as plsc

N = 10000
E = 320000
D = 128
H = 128
C = 64

N_PAD = 10112
ROW_BLK = 1264

NC = 2
NS = 16
BLK = 128
PH = 20
P0_L1, P1_L1 = 6, 2
P0_L2, P1_L2 = 6, 2
EPP = NS * PH * BLK
E_PAD = 8 * EPP
STRIPE = N_PAD // NS


def _stage_a_body(x_ref, w1lt_ref, w1rt_ref, xt1_ref, xr1_ref):
    x = x_ref[...]
    xt1_ref[...] = jnp.dot(x, w1lt_ref[...], preferred_element_type=jnp.float32)
    xr1_ref[...] = jnp.dot(x, w1rt_ref[...], preferred_element_type=jnp.float32)


def _stage_b_body(s1a_ref, s1b_ref, cnta_ref, cntb_ref, xr1_ref, b1l_ref,
                  w2lt_ref, w2rt_ref, ht2_ref, hr2_ref):
    cnt = cnta_ref[:, 0:1] + cntb_ref[:, 0:1]
    den = jnp.maximum(cnt, 1.0)
    h = (s1a_ref[...] + s1b_ref[...]) / den + b1l_ref[...] + xr1_ref[...]
    h = jnp.maximum(h, 0.0)
    ht2_ref[...] = jnp.dot(h, w2lt_ref[...], preferred_element_type=jnp.float32)
    hr2_ref[...] = jnp.dot(h, w2rt_ref[...], preferred_element_type=jnp.float32)


def _stage_c_body(s2a_ref, s2b_ref, cnta_ref, cntb_ref, hr2_ref, b2l_ref, out_ref):
    cnt = cnta_ref[:, 0:1] + cntb_ref[:, 0:1]
    den = jnp.maximum(cnt, 1.0)
    o = (s2a_ref[...] + s2b_ref[...]) / den + b2l_ref[...] + hr2_ref[...]
    m = jnp.max(o, axis=1, keepdims=True)
    ex = jnp.exp(o - m)
    lse = m + jnp.log(jnp.sum(ex, axis=1, keepdims=True))
    out_ref[...] = o - lse


def _row_spec(w):
    return pl.BlockSpec((ROW_BLK, w), lambda i: (i, 0))


def _full_spec(h, w):
    return pl.BlockSpec((h, w), lambda i: (0, 0))


def _stage_a(x_pad, w1lt, w1rt):
    return pl.pallas_call(
        _stage_a_body,
        grid=(N_PAD // ROW_BLK,),
        in_specs=[_row_spec(D), _full_spec(D, H), _full_spec(D, H)],
        out_specs=[_row_spec(H), _row_spec(H)],
        out_shape=[jax.ShapeDtypeStruct((N_PAD, H), jnp.float32)] * 2,
    )(x_pad, w1lt, w1rt)


def _stage_b(s1a, s1b, cnta, cntb, xr1, b1l, w2lt, w2rt):
    return pl.pallas_call(
        _stage_b_body,
        grid=(N_PAD // ROW_BLK,),
        in_specs=[_row_spec(H), _row_spec(H), _row_spec(16), _row_spec(16),
                  _row_spec(H), _full_spec(1, H), _full_spec(H, C), _full_spec(H, C)],
        out_specs=[_row_spec(C), _row_spec(C)],
        out_shape=[jax.ShapeDtypeStruct((N_PAD, C), jnp.float32)] * 2,
    )(s1a, s1b, cnta, cntb, xr1, b1l, w2lt, w2rt)


def _stage_c(s2a, s2b, cnta, cntb, hr2, b2l):
    return pl.pallas_call(
        _stage_c_body,
        grid=(N_PAD // ROW_BLK,),
        in_specs=[_row_spec(C), _row_spec(C), _row_spec(16), _row_spec(16),
                  _row_spec(C), _full_spec(1, C)],
        out_specs=_row_spec(C),
        out_shape=jax.ShapeDtypeStruct((N_PAD, C), jnp.float32),
    )(s2a, s2b, cnta, cntb, hr2, b2l)


def _make_sc_seg_sum(width, with_cnt, p0, p1):
    mesh = plsc.VectorSubcoreMesh(core_axis_name="c", subcore_axis_name="s",
                                  num_cores=NC, num_subcores=NS)
    out_type = [jax.ShapeDtypeStruct((N_PAD, width), jnp.float32)] * NC
    if with_cnt:
        out_type += [jax.ShapeDtypeStruct((N_PAD, 16), jnp.float32)] * NC
    scratch = [
        pltpu.VMEM((PH, BLK), jnp.int32),
        pltpu.VMEM((PH, BLK), jnp.int32),
        pltpu.VMEM((BLK, width), jnp.float32),
        pltpu.VMEM((BLK, width), jnp.float32),
        pltpu.SemaphoreType.DMA,
        pltpu.SemaphoreType.DMA,
        pltpu.SemaphoreType.DMA,
        pltpu.SemaphoreType.DMA,
        pltpu.SemaphoreType.DMA,
        pltpu.VMEM_SHARED((N_PAD, width), jnp.float32),
    ]
    if with_cnt:
        scratch += [
            pltpu.VMEM((BLK, 16), jnp.float32),
            pltpu.VMEM_SHARED((N_PAD, 16), jnp.float32),
        ]

    def body(*refs):
        ni = 4 if p1 else 2
        if with_cnt:
            table = refs[0]
            idxs = refs[1:1 + ni]
            (zerosw, zeros16, ones16) = refs[1 + ni:4 + ni]
            outs = refs[4 + ni:4 + ni + NC]
            cnts = refs[4 + ni + NC:4 + ni + 2 * NC]
            (src_v, dst_v, rows0, rows1, g0, g1, s0, s1, sc_sem,
             accum_sh, ones_v, cnt_sh) = refs[4 + ni + 2 * NC:]
        else:
            table = refs[0]
            idxs = refs[1:1 + ni]
            zerosw = refs[1 + ni]
            outs = refs[2 + ni:2 + ni + NC]
            (src_v, dst_v, rows0, rows1, g0, g1, s0, s1, sc_sem,
             accum_sh) = refs[2 + ni + NC:]
        if p1:
            srcA, srcB, dstA, dstB = idxs
        else:
            (srcA, dstA), srcB, dstB = idxs, None, None
        cid = lax.axis_index("c")
        sid = lax.axis_index("s")
        base = sid * STRIPE
        stripe = pl.ds(base, STRIPE)

        pltpu.sync_copy(zerosw.at[stripe], accum_sh.at[stripe])
        if with_cnt:
            pltpu.sync_copy(ones16, ones_v)
            pltpu.sync_copy(zeros16.at[stripe], cnt_sh.at[stripe])
        plsc.subcore_barrier()

        rows = (rows0, rows1)
        gsem = (g0, g1)
        ssem = (s0, s1)

        def gather(j, b):
            pltpu.async_copy(table.at[src_v.at[j]], rows[b], gsem[b])

        def wait_gather(j, b):
            pltpu.make_async_copy(table.at[src_v.at[j]], rows[b], gsem[b]).wait()

        def scatter(j, b):
            pltpu.async_copy(rows[b], accum_sh.at[dst_v.at[j]], ssem[b], add=True)
            if with_cnt:
                pltpu.async_copy(ones_v, cnt_sh.at[dst_v.at[j]], sc_sem, add=True)

        def wait_scatter(j, b):
            pltpu.make_async_copy(rows[b], accum_sh.at[dst_v.at[j]], ssem[b]).wait()

        def wait_cnt(j):
            if with_cnt:
                pltpu.make_async_copy(ones_v, cnt_sh.at[dst_v.at[j]], sc_sem).wait()

        def phase(srcr, dstr, p):
            pltpu.sync_copy(srcr.at[sid, pl.ds(p * PH, PH)], src_v)
            pltpu.sync_copy(dstr.at[sid, pl.ds(p * PH, PH)], dst_v)
            gather(0, 0)

            def pair(q, carry):
                j = 2 * q

                @pl.when(q > 0)
                def _():
                    wait_scatter(j - 1, 1)
                    wait_cnt(j - 2)
                    wait_cnt(j - 1)
                gather(j + 1, 1)
                wait_gather(j, 0)
                scatter(j, 0)

                @pl.when(q < PH // 2 - 1)
                def _():
                    wait_scatter(j, 0)
                    gather(j + 2, 0)
                wait_gather(j + 1, 1)
                scatter(j + 1, 1)
                return carry

            lax.fori_loop(0, PH // 2, pair, 0)
            wait_scatter(PH - 2, 0)
            wait_cnt(PH - 2)
            wait_scatter(PH - 1, 1)
            wait_cnt(PH - 1)

        @pl.when(cid == 0)
        def _():
            for p in range(p0):
                phase(srcA, dstA, p)

        if p1:
            @pl.when(cid == 1)
            def _():
                for p in range(p1):
                    phase(srcB, dstB, p)
        plsc.subcore_barrier()

        for c in range(NC):
            @pl.when(cid == c)
            def _(c=c):
                pltpu.sync_copy(accum_sh.at[stripe], outs[c].at[stripe])
                if with_cnt:
                    pltpu.sync_copy(cnt_sh.at[stripe], cnts[c].at[stripe])

    return pl.kernel(
        body, out_type=out_type, mesh=mesh, scratch_types=scratch,
        compiler_params=pltpu.CompilerParams(use_tc_tiling_on_sc=False))


_seg_sum_l1 = _make_sc_seg_sum(H, True, P0_L1, P1_L1)
_seg_sum_l2 = _make_sc_seg_sum(C, False, P0_L2, P1_L2)


def _split_edges(arr, fill, p0, p1):
    e0 = min(NS * p0 * PH * BLK, E)
    pad0 = NS * p0 * PH * BLK - e0
    a = jnp.concatenate([arr[:e0], jnp.full((pad0,), fill, jnp.int32)])
    a = a.reshape(NS, p0 * PH, BLK)
    if p1 == 0:
        return (a,)
    pad1 = NS * p1 * PH * BLK - (E - e0)
    b = jnp.concatenate([arr[e0:], jnp.full((pad1,), fill, jnp.int32)])
    return a, b.reshape(NS, p1 * PH, BLK)


def kernel(x, edge_index, W1l, b1l, W1r, W2l, b2l, W2r):
    src = edge_index[0]
    dst = edge_index[1]

    s1_idx = _split_edges(src, 0, P0_L1, P1_L1) + _split_edges(dst, N, P0_L1, P1_L1)
    s2_idx = _split_edges(src, 0, P0_L2, P1_L2) + _split_edges(dst, N, P0_L2, P1_L2)

    x_pad = jnp.zeros((N_PAD, D), jnp.float32).at[:N].set(x)
    w1lt = W1l.T
    w1rt = W1r.T
    w2lt = W2l.T
    w2rt = W2r.T

    zeros_h = jnp.zeros((N_PAD, H), jnp.float32)
    zeros_c = jnp.zeros((N_PAD, C), jnp.float32)
    zeros16 = jnp.zeros((N_PAD, 16), jnp.float32)
    ones16 = jnp.ones((BLK, 16), jnp.float32)

    xt1, xr1 = _stage_a(x_pad, w1lt, w1rt)

    r1 = _seg_sum_l1(xt1, *s1_idx, zeros_h, zeros16, ones16)
    if NC == 2:
        s1a, s1b, cnta, cntb = r1
    else:
        (s1a, cnta), s1b, cntb = r1, zeros_h, zeros16

    ht2, hr2 = _stage_b(s1a, s1b, cnta, cntb, xr1, b1l.reshape(1, H), w2lt, w2rt)

    r2 = _seg_sum_l2(ht2, *s2_idx, zeros_c)
    if NC == 2:
        s2a, s2b = r2
    else:
        (s2a,), s2b = r2, zeros_c

    out = _stage_c(s2a, s2b, cnta, cntb, hr2, b2l.reshape(1, C))
    return out[:N]
